# baseline (speedup 1.0000x reference)
import jax
import jax.numpy as jnp
from jax import lax
from jax.experimental import pallas as pl
from jax.experimental.pallas import tpu as pltpu
from jax.experimental.pallas import tpu_sc as plsc

B, N, F = 16, 200, 32
NN = N * N
L = 16
PAIR_V = 25


def _tuples_body(x_hbm, out_hbm, xb, pbuf, sems):
  wid = lax.axis_index("s") * 2 + lax.axis_index("c")
  b = wid // 2
  sub = wid % 2

  pltpu.sync_copy(x_hbm.at[pl.ds(b, 1)], xb)

  lanes = lax.iota(jnp.int32, L)
  lo8 = lanes < 8

  def fill_step_plane(c, s):
    cvec = jnp.full((L,), c, jnp.int32)
    zvec = jnp.zeros((L,), jnp.int32)

    def pair(p, carry):
      base = p * 400
      v0 = plsc.load_gather(xb, [zvec, jnp.full((L,), 2 * p, jnp.int32), cvec])
      v1 = plsc.load_gather(
          xb, [zvec, jnp.full((L,), 2 * p + 1, jnp.int32), cvec]
      )
      vm = jnp.where(lo8, v0, v1)
      for u in range(12):
        pbuf[s, 0, pl.ds(base + u * L, L)] = v0
      pbuf[s, 0, pl.ds(base + 192, L)] = vm
      for u in range(12):
        pbuf[s, 0, pl.ds(base + 208 + u * L, L)] = v1
      return carry

    lax.fori_loop(0, N // 2, pair, 0)

  def fill_tiled_plane(c, s):
    cvec = jnp.full((L,), c - F, jnp.int32)
    zvec = jnp.zeros((L,), jnp.int32)
    pat = []
    for k in range(PAIR_V):
      rvec = lanes + (k * L) % N
      rvec = jnp.where(rvec >= N, rvec - N, rvec)
      pat.append(plsc.load_gather(xb, [zvec, rvec, cvec]))

    def rep(p, carry):
      base = p * 400
      for u in range(PAIR_V):
        pbuf[s, 0, pl.ds(base + u * L, L)] = pat[u]
      return carry

    lax.fori_loop(0, N // 2, rep, 0)

  def plane_pair(m, carry):
    c_step = sub * 16 + m
    c_tile = F + sub * 16 + m

    @pl.when(m >= 1)
    def _():
      pltpu.make_async_copy(
          pbuf.at[pl.ds(0, 1)],
          out_hbm.at[pl.ds(b, 1), pl.ds(c_step, 1)],
          sems.at[0],
      ).wait()

    fill_step_plane(c_step, 0)
    pltpu.async_copy(
        pbuf.at[pl.ds(0, 1)],
        out_hbm.at[pl.ds(b, 1), pl.ds(c_step, 1)],
        sems.at[0],
    )

    @pl.when(m >= 1)
    def _():
      pltpu.make_async_copy(
          pbuf.at[pl.ds(1, 1)],
          out_hbm.at[pl.ds(b, 1), pl.ds(c_tile, 1)],
          sems.at[1],
      ).wait()

    fill_tiled_plane(c_tile, 1)
    pltpu.async_copy(
        pbuf.at[pl.ds(1, 1)],
        out_hbm.at[pl.ds(b, 1), pl.ds(c_tile, 1)],
        sems.at[1],
    )
    return carry

  lax.fori_loop(0, 16, plane_pair, 0)

  for k in range(2):
    pltpu.make_async_copy(
        pbuf.at[pl.ds(k, 1)],
        out_hbm.at[pl.ds(b, 1), pl.ds(0, 1)],
        sems.at[k],
    ).wait()


_tuples_sc = pl.kernel(
    _tuples_body,
    out_type=jax.ShapeDtypeStruct((B, 2 * F, NN), jnp.float32),
    mesh=plsc.VectorSubcoreMesh(
        core_axis_name="c", subcore_axis_name="s", num_cores=2, num_subcores=16
    ),
    compiler_params=pltpu.CompilerParams(needs_layout_passes=False),
    scratch_types=[
        pltpu.VMEM((1, N, F), jnp.float32),
        pltpu.VMEM((2, 1, NN), jnp.float32),
        pltpu.SemaphoreType.DMA((2,)),
    ],
)


@jax.jit
def kernel(x):
  return _tuples_sc(x).transpose(0, 2, 1)


# speedup vs baseline: 1.0019x; 1.0019x over previous
import jax
import jax.numpy as jnp
from jax import lax
from jax.experimental import pallas as pl
from jax.experimental.pallas import tpu as pltpu
from jax.experimental.pallas import tpu_sc as plsc

B, N, F = 16, 200, 32
NN = N * N
L = 16
PAIR_V = 25


def _tuples_body(x_hbm, out_hbm, xb, pbuf, sems):
  wid = lax.axis_index("s") * 2 + lax.axis_index("c")
  b = wid // 2
  sub = wid % 2

  pltpu.sync_copy(x_hbm.at[pl.ds(b, 1)], xb)

  lanes = lax.iota(jnp.int32, L)
  lo8 = lanes < 8

  def fill_step_plane(c, s):
    cvec = jnp.full((L,), c, jnp.int32)
    zvec = jnp.zeros((L,), jnp.int32)

    def pair(p, carry):
      base = p * 400
      v0 = plsc.load_gather(xb, [zvec, jnp.full((L,), 2 * p, jnp.int32), cvec])
      v1 = plsc.load_gather(
          xb, [zvec, jnp.full((L,), 2 * p + 1, jnp.int32), cvec]
      )
      vm = jnp.where(lo8, v0, v1)
      for u in range(12):
        pbuf[s, 0, pl.ds(base + u * L, L)] = v0
      pbuf[s, 0, pl.ds(base + 192, L)] = vm
      for u in range(12):
        pbuf[s, 0, pl.ds(base + 208 + u * L, L)] = v1
      return carry

    lax.fori_loop(0, N // 2, pair, 0)

  def fill_tiled_plane(c, s):
    cvec = jnp.full((L,), c - F, jnp.int32)
    zvec = jnp.zeros((L,), jnp.int32)
    pat = []
    for k in range(PAIR_V):
      rvec = lanes + (k * L) % N
      rvec = jnp.where(rvec >= N, rvec - N, rvec)
      pat.append(plsc.load_gather(xb, [zvec, rvec, cvec]))

    def rep(p, carry):
      base = p * 400
      for u in range(PAIR_V):
        pbuf[s, 0, pl.ds(base + u * L, L)] = pat[u]
      return carry

    lax.fori_loop(0, N // 2, rep, 0)

  def plane_pair(m, carry):
    c_step = sub * 16 + m
    c_tile = F + sub * 16 + m

    @pl.when(m >= 1)
    def _():
      pltpu.make_async_copy(
          pbuf.at[pl.ds(0, 1)],
          out_hbm.at[pl.ds(b, 1), pl.ds(c_step, 1)],
          sems.at[0],
      ).wait()

    fill_step_plane(c_step, 0)
    pltpu.async_copy(
        pbuf.at[pl.ds(0, 1)],
        out_hbm.at[pl.ds(b, 1), pl.ds(c_step, 1)],
        sems.at[0],
    )

    @pl.when(m >= 1)
    def _():
      pltpu.make_async_copy(
          pbuf.at[pl.ds(1, 1)],
          out_hbm.at[pl.ds(b, 1), pl.ds(c_tile, 1)],
          sems.at[1],
      ).wait()

    fill_tiled_plane(c_tile, 1)
    pltpu.async_copy(
        pbuf.at[pl.ds(1, 1)],
        out_hbm.at[pl.ds(b, 1), pl.ds(c_tile, 1)],
        sems.at[1],
    )
    return carry

  lax.fori_loop(0, 16, plane_pair, 0)

  for k in range(2):
    pltpu.make_async_copy(
        pbuf.at[pl.ds(k, 1)],
        out_hbm.at[pl.ds(b, 1), pl.ds(0, 1)],
        sems.at[k],
    ).wait()


_tuples_sc = pl.kernel(
    _tuples_body,
    out_type=jax.ShapeDtypeStruct((B, 2 * F, NN), jnp.float32),
    mesh=plsc.VectorSubcoreMesh(
        core_axis_name="c", subcore_axis_name="s", num_cores=2, num_subcores=16
    ),
    compiler_params=pltpu.CompilerParams(
        needs_layout_passes=False, use_tc_tiling_on_sc=True
    ),
    scratch_types=[
        pltpu.VMEM((1, N, F), jnp.float32),
        pltpu.VMEM((2, 1, NN), jnp.float32),
        pltpu.SemaphoreType.DMA((2,)),
    ],
)


@jax.jit
def kernel(x):
  return _tuples_sc(x).transpose(0, 2, 1)
